# baseline (speedup 1.0000x reference)
import functools

import jax
import jax.numpy as jnp
from jax import lax
from jax.experimental import pallas as pl
from jax.experimental.pallas import tpu as pltpu
from jax.experimental.pallas import tpu_sc as plsc

N = 10000
E = 320000
D = 128
ODIM = 16

NC = 2
NS = 16
EPT = E // NS
CHUNK = 80
NCH = EPT // CHUNK
NPAD = 10240
RPT = NPAD // NS
GRP = 25
NGRP = NCH // GRP
RB = 1000


def _node_tables(h, nt, wfr_ref, wbe_ref, war_ref, bar_ref, wself_ref,
                 bself_ref, base_ref, diff_ref, selfo_ref, alpha_ref):
    hfr = jnp.dot(h, wfr_ref[...], preferred_element_type=jnp.float32)
    hbe = jnp.dot(h, wbe_ref[...], preferred_element_type=jnp.float32)
    selfo_ref[...] = (
        jnp.dot(h, wself_ref[...], preferred_element_type=jnp.float32)
        + bself_ref[...])
    a_pre = jnp.sum(h * war_ref[...], axis=1, keepdims=True) + bar_ref[...][:, :1]
    a = jax.nn.sigmoid(a_pre)
    base_ref[...] = jnp.where(nt == 1, hfr, hbe)
    diff_ref[...] = jnp.where(nt == 2, hfr - hbe, jnp.zeros_like(hfr))
    alpha_ref[...] = jnp.broadcast_to(a, (h.shape[0], D))


def _dense_in_body(x_ref, nt_ref, wfr_ref, wbe_ref, war_ref, bar_ref,
                   wself_ref, bself_ref, base_ref, diff_ref, selfo_ref,
                   alpha_ref):
    _node_tables(x_ref[...], nt_ref[...], wfr_ref, wbe_ref, war_ref, bar_ref,
                 wself_ref, bself_ref, base_ref, diff_ref, selfo_ref, alpha_ref)


def _dense_mid_body(a_ref, b_ref, al_ref, so_ref, nt_ref, wfr_ref, wbe_ref,
                    war_ref, bar_ref, wself_ref, bself_ref, base_ref,
                    diff_ref, selfo_ref, alpha_ref):
    h = jnp.maximum(a_ref[...] + al_ref[...] * b_ref[...] + so_ref[...], 0.0)
    _node_tables(h, nt_ref[...], wfr_ref, wbe_ref, war_ref, bar_ref,
                 wself_ref, bself_ref, base_ref, diff_ref, selfo_ref, alpha_ref)


def _dense_out_body(a_ref, b_ref, al_ref, so_ref, wc_ref, bc_ref, out_ref):
    h = jnp.maximum(a_ref[...] + al_ref[...] * b_ref[...] + so_ref[...], 0.0)
    out_ref[...] = (
        jnp.dot(h, wc_ref[...], preferred_element_type=jnp.float32)
        + bc_ref[...])


_rowmap = lambda i: (i, 0)
_fixmap = lambda i: (0, 0)


def _dense_in(x, nt, wfr, wbe, war, bar, wself, bselfr):
    return pl.pallas_call(
        _dense_in_body,
        grid=(N // RB,),
        in_specs=[
            pl.BlockSpec((RB, D), _rowmap),
            pl.BlockSpec((RB, 1), _rowmap),
            pl.BlockSpec((D, D), _fixmap),
            pl.BlockSpec((D, D), _fixmap),
            pl.BlockSpec((1, D), _fixmap),
            pl.BlockSpec((1, D), _fixmap),
            pl.BlockSpec((D, D), _fixmap),
            pl.BlockSpec((1, D), _fixmap),
        ],
        out_specs=[pl.BlockSpec((RB, D), _rowmap)] * 4,
        out_shape=[jax.ShapeDtypeStruct((N, D), jnp.float32)] * 4,
    )(x, nt, wfr, wbe, war, bar, wself, bselfr)


def _dense_mid(a, b, al, so, nt, wfr, wbe, war, bar, wself, bselfr):
    return pl.pallas_call(
        _dense_mid_body,
        grid=(N // RB,),
        in_specs=[
            pl.BlockSpec((RB, D), _rowmap),
            pl.BlockSpec((RB, D), _rowmap),
            pl.BlockSpec((RB, D), _rowmap),
            pl.BlockSpec((RB, D), _rowmap),
            pl.BlockSpec((RB, 1), _rowmap),
            pl.BlockSpec((D, D), _fixmap),
            pl.BlockSpec((D, D), _fixmap),
            pl.BlockSpec((1, D), _fixmap),
            pl.BlockSpec((1, D), _fixmap),
            pl.BlockSpec((D, D), _fixmap),
            pl.BlockSpec((1, D), _fixmap),
        ],
        out_specs=[pl.BlockSpec((RB, D), _rowmap)] * 4,
        out_shape=[jax.ShapeDtypeStruct((N, D), jnp.float32)] * 4,
    )(a, b, al, so, nt, wfr, wbe, war, bar, wself, bselfr)


def _dense_out(a, b, al, so, wc, bcr):
    return pl.pallas_call(
        _dense_out_body,
        grid=(N // RB,),
        in_specs=[
            pl.BlockSpec((RB, D), _rowmap),
            pl.BlockSpec((RB, D), _rowmap),
            pl.BlockSpec((RB, D), _rowmap),
            pl.BlockSpec((RB, D), _rowmap),
            pl.BlockSpec((D, ODIM), _fixmap),
            pl.BlockSpec((1, ODIM), _fixmap),
        ],
        out_specs=pl.BlockSpec((RB, ODIM), _rowmap),
        out_shape=jax.ShapeDtypeStruct((N, ODIM), jnp.float32),
    )(a, b, al, so, wc, bcr)


def _sc_seg_body(base_hbm, diff_hbm, src_hbm, dst_hbm, zeros_hbm, a_hbm, b_hbm,
                 sidx, didx, rows0, rows1, acc, semg0, semg1, sema0, sema1):
    c = lax.axis_index("c")
    s = lax.axis_index("s")

    pltpu.sync_copy(zeros_hbm, acc.at[pl.ds(s * RPT, RPT)])
    plsc.subcore_barrier()

    def run(tab_hbm):
        def wait_g(rows, semg):
            pltpu.make_async_copy(tab_hbm.at[sidx.at[0]], rows, semg).wait()

        def wait_a(rows, sema):
            pltpu.make_async_copy(rows, acc.at[didx.at[0]], sema).wait()

        @pl.loop(0, NGRP)
        def _(g):
            pltpu.sync_copy(src_hbm.at[s, g], sidx)
            pltpu.sync_copy(dst_hbm.at[s, g], didx)

            pltpu.async_copy(tab_hbm.at[sidx.at[0]], rows0, semg0)
            pltpu.async_copy(tab_hbm.at[sidx.at[1]], rows1, semg1)

            @pl.loop(0, GRP - 1, step=2)
            def _(k):
                wait_g(rows0, semg0)
                pltpu.async_copy(rows0, acc.at[didx.at[k]], sema0, add=True)
                wait_g(rows1, semg1)
                pltpu.async_copy(rows1, acc.at[didx.at[k + 1]], sema1, add=True)
                wait_a(rows0, sema0)
                pltpu.async_copy(tab_hbm.at[sidx.at[k + 2]], rows0, semg0)
                wait_a(rows1, sema1)

                @pl.when(k + 3 < GRP)
                def _():
                    pltpu.async_copy(tab_hbm.at[sidx.at[k + 3]], rows1, semg1)

            wait_g(rows0, semg0)
            pltpu.async_copy(rows0, acc.at[didx.at[GRP - 1]], sema0, add=True)
            wait_a(rows0, sema0)

    @pl.when(c == 0)
    def _():
        run(base_hbm)

    @pl.when(c == 1)
    def _():
        run(diff_hbm)

    plsc.subcore_barrier()

    @pl.when(c == 0)
    def _():
        pltpu.sync_copy(acc.at[pl.ds(s * RPT, RPT)],
                        a_hbm.at[pl.ds(s * RPT, RPT)])

    @pl.when(c == 1)
    def _():
        pltpu.sync_copy(acc.at[pl.ds(s * RPT, RPT)],
                        b_hbm.at[pl.ds(s * RPT, RPT)])


@functools.cache
def _sc_seg_build():
    mesh = plsc.VectorSubcoreMesh(core_axis_name="c", subcore_axis_name="s",
                                  num_cores=NC, num_subcores=NS)
    return pl.kernel(
        _sc_seg_body,
        out_type=(jax.ShapeDtypeStruct((NPAD, D), jnp.float32),
                  jax.ShapeDtypeStruct((NPAD, D), jnp.float32)),
        mesh=mesh,
        scratch_types=[
            pltpu.VMEM((GRP, CHUNK), jnp.int32),
            pltpu.VMEM((GRP, CHUNK), jnp.int32),
            pltpu.VMEM((CHUNK, D), jnp.float32),
            pltpu.VMEM((CHUNK, D), jnp.float32),
            pltpu.VMEM_SHARED((NPAD, D), jnp.float32),
            pltpu.SemaphoreType.DMA,
            pltpu.SemaphoreType.DMA,
            pltpu.SemaphoreType.DMA,
            pltpu.SemaphoreType.DMA,
        ],
    )


def _sc_seg(base, diff, src, dst, zeros):
    return _sc_seg_build()(base, diff, src, dst, zeros)


def kernel(x, edge_index, y, pmp_mask,
           W_fr1, W_be1, Wa1, ba1, W_self1, b_self1,
           W_fr2, W_be2, Wa2, ba2, W_self2, b_self2,
           Wc, bc):
    nt = jnp.where(pmp_mask, y.astype(jnp.int32), 2).astype(jnp.int32)[:, None]
    src = edge_index[0].astype(jnp.int32).reshape(NS, NGRP, GRP, CHUNK)
    dst = edge_index[1].astype(jnp.int32).reshape(NS, NGRP, GRP, CHUNK)
    zeros = jnp.zeros((RPT, D), jnp.float32)

    wa1r = Wa1.reshape(1, D)
    ba1r = jnp.broadcast_to(ba1.reshape(1, 1), (1, D))
    wa2r = Wa2.reshape(1, D)
    ba2r = jnp.broadcast_to(ba2.reshape(1, 1), (1, D))
    bs1r = b_self1.reshape(1, D)
    bs2r = b_self2.reshape(1, D)
    bcr = bc.reshape(1, ODIM)

    base1, diff1, self1, alpha1 = _dense_in(
        x, nt, W_fr1, W_be1, wa1r, ba1r, W_self1, bs1r)
    A1, B1 = _sc_seg(base1, diff1, src, dst, zeros)
    base2, diff2, self2, alpha2 = _dense_mid(
        A1, B1, alpha1, self1, nt, W_fr2, W_be2, wa2r, ba2r, W_self2, bs2r)
    A2, B2 = _sc_seg(base2, diff2, src, dst, zeros)
    return _dense_out(A2, B2, alpha2, self2, Wc, bcr)


# speedup vs baseline: 1.2866x; 1.2866x over previous
import functools

import jax
import jax.numpy as jnp
from jax import lax
from jax.experimental import pallas as pl
from jax.experimental.pallas import tpu as pltpu
from jax.experimental.pallas import tpu_sc as plsc

N = 10000
E = 320000
D = 128
ODIM = 16

NC = 2
NS = 16
EPT = E // NS
CHUNK = 128
GRP = 12
NGRP = 13
TAIL = EPT - NGRP * GRP * CHUNK
NPAD = 10240
RPT = NPAD // NS
RB = 1000


def _node_tables(h, nt, wfr_ref, wbe_ref, war_ref, bar_ref, wself_ref,
                 bself_ref, base_ref, diff_ref, selfo_ref, alpha_ref):
    hfr = jnp.dot(h, wfr_ref[...], preferred_element_type=jnp.float32)
    hbe = jnp.dot(h, wbe_ref[...], preferred_element_type=jnp.float32)
    selfo_ref[...] = (
        jnp.dot(h, wself_ref[...], preferred_element_type=jnp.float32)
        + bself_ref[...])
    a_pre = jnp.sum(h * war_ref[...], axis=1, keepdims=True) + bar_ref[...][:, :1]
    a = jax.nn.sigmoid(a_pre)
    base_ref[...] = jnp.where(nt == 1, hfr, hbe)
    diff_ref[...] = jnp.where(nt == 2, hfr - hbe, jnp.zeros_like(hfr))
    alpha_ref[...] = jnp.broadcast_to(a, (h.shape[0], D))


def _dense_in_body(x_ref, nt_ref, wfr_ref, wbe_ref, war_ref, bar_ref,
                   wself_ref, bself_ref, base_ref, diff_ref, selfo_ref,
                   alpha_ref):
    _node_tables(x_ref[...], nt_ref[...], wfr_ref, wbe_ref, war_ref, bar_ref,
                 wself_ref, bself_ref, base_ref, diff_ref, selfo_ref, alpha_ref)


def _dense_mid_body(a_ref, b_ref, al_ref, so_ref, nt_ref, wfr_ref, wbe_ref,
                    war_ref, bar_ref, wself_ref, bself_ref, base_ref,
                    diff_ref, selfo_ref, alpha_ref):
    h = jnp.maximum(a_ref[...] + al_ref[...] * b_ref[...] + so_ref[...], 0.0)
    _node_tables(h, nt_ref[...], wfr_ref, wbe_ref, war_ref, bar_ref,
                 wself_ref, bself_ref, base_ref, diff_ref, selfo_ref, alpha_ref)


def _dense_out_body(a_ref, b_ref, al_ref, so_ref, wc_ref, bc_ref, out_ref):
    h = jnp.maximum(a_ref[...] + al_ref[...] * b_ref[...] + so_ref[...], 0.0)
    out_ref[...] = (
        jnp.dot(h, wc_ref[...], preferred_element_type=jnp.float32)
        + bc_ref[...])


_rowmap = lambda i: (i, 0)
_fixmap = lambda i: (0, 0)


def _dense_in(x, nt, wfr, wbe, war, bar, wself, bselfr):
    return pl.pallas_call(
        _dense_in_body,
        grid=(N // RB,),
        in_specs=[
            pl.BlockSpec((RB, D), _rowmap),
            pl.BlockSpec((RB, 1), _rowmap),
            pl.BlockSpec((D, D), _fixmap),
            pl.BlockSpec((D, D), _fixmap),
            pl.BlockSpec((1, D), _fixmap),
            pl.BlockSpec((1, D), _fixmap),
            pl.BlockSpec((D, D), _fixmap),
            pl.BlockSpec((1, D), _fixmap),
        ],
        out_specs=[pl.BlockSpec((RB, D), _rowmap)] * 4,
        out_shape=[jax.ShapeDtypeStruct((N, D), jnp.float32)] * 4,
    )(x, nt, wfr, wbe, war, bar, wself, bselfr)


def _dense_mid(a, b, al, so, nt, wfr, wbe, war, bar, wself, bselfr):
    return pl.pallas_call(
        _dense_mid_body,
        grid=(N // RB,),
        in_specs=[
            pl.BlockSpec((RB, D), _rowmap),
            pl.BlockSpec((RB, D), _rowmap),
            pl.BlockSpec((RB, D), _rowmap),
            pl.BlockSpec((RB, D), _rowmap),
            pl.BlockSpec((RB, 1), _rowmap),
            pl.BlockSpec((D, D), _fixmap),
            pl.BlockSpec((D, D), _fixmap),
            pl.BlockSpec((1, D), _fixmap),
            pl.BlockSpec((1, D), _fixmap),
            pl.BlockSpec((D, D), _fixmap),
            pl.BlockSpec((1, D), _fixmap),
        ],
        out_specs=[pl.BlockSpec((RB, D), _rowmap)] * 4,
        out_shape=[jax.ShapeDtypeStruct((N, D), jnp.float32)] * 4,
    )(a, b, al, so, nt, wfr, wbe, war, bar, wself, bselfr)


def _dense_out(a, b, al, so, wc, bcr):
    return pl.pallas_call(
        _dense_out_body,
        grid=(N // RB,),
        in_specs=[
            pl.BlockSpec((RB, D), _rowmap),
            pl.BlockSpec((RB, D), _rowmap),
            pl.BlockSpec((RB, D), _rowmap),
            pl.BlockSpec((RB, D), _rowmap),
            pl.BlockSpec((D, ODIM), _fixmap),
            pl.BlockSpec((1, ODIM), _fixmap),
        ],
        out_specs=pl.BlockSpec((RB, ODIM), _rowmap),
        out_shape=jax.ShapeDtypeStruct((N, ODIM), jnp.float32),
    )(a, b, al, so, wc, bcr)


def _sc_seg_body(base_hbm, diff_hbm, src_hbm, dst_hbm, srct_hbm, dstt_hbm,
                 zeros_hbm, a_hbm, b_hbm,
                 sidx, didx, sidxt, didxt, rows0, rows1, rowst, acc,
                 semg0, semg1):
    c = lax.axis_index("c")
    s = lax.axis_index("s")

    pltpu.sync_copy(zeros_hbm, acc.at[pl.ds(s * RPT, RPT)])
    plsc.subcore_barrier()

    def run(tab_hbm):
        def wait_g(rows, semg):
            pltpu.make_async_copy(tab_hbm.at[sidx.at[0]], rows, semg).wait()

        @pl.loop(0, NGRP)
        def _(g):
            pltpu.sync_copy(src_hbm.at[s, g], sidx)
            pltpu.sync_copy(dst_hbm.at[s, g], didx)

            pltpu.async_copy(tab_hbm.at[sidx.at[0]], rows0, semg0)

            @pl.loop(0, GRP - 2, step=2)
            def _(k):
                pltpu.async_copy(tab_hbm.at[sidx.at[k + 1]], rows1, semg1)
                wait_g(rows0, semg0)
                pltpu.sync_copy(rows0, acc.at[didx.at[k]], add=True)
                pltpu.async_copy(tab_hbm.at[sidx.at[k + 2]], rows0, semg0)
                wait_g(rows1, semg1)
                pltpu.sync_copy(rows1, acc.at[didx.at[k + 1]], add=True)

            pltpu.async_copy(tab_hbm.at[sidx.at[GRP - 1]], rows1, semg1)
            wait_g(rows0, semg0)
            pltpu.sync_copy(rows0, acc.at[didx.at[GRP - 2]], add=True)
            wait_g(rows1, semg1)
            pltpu.sync_copy(rows1, acc.at[didx.at[GRP - 1]], add=True)

        pltpu.sync_copy(srct_hbm.at[s], sidxt)
        pltpu.sync_copy(dstt_hbm.at[s], didxt)
        pltpu.async_copy(tab_hbm.at[sidxt.at[0]], rowst, semg0).wait()
        pltpu.sync_copy(rowst, acc.at[didxt.at[0]], add=True)

    @pl.when(c == 0)
    def _():
        run(base_hbm)

    @pl.when(c == 1)
    def _():
        run(diff_hbm)

    plsc.subcore_barrier()

    @pl.when(c == 0)
    def _():
        pltpu.sync_copy(acc.at[pl.ds(s * RPT, RPT)],
                        a_hbm.at[pl.ds(s * RPT, RPT)])

    @pl.when(c == 1)
    def _():
        pltpu.sync_copy(acc.at[pl.ds(s * RPT, RPT)],
                        b_hbm.at[pl.ds(s * RPT, RPT)])


@functools.cache
def _sc_seg_build():
    mesh = plsc.VectorSubcoreMesh(core_axis_name="c", subcore_axis_name="s",
                                  num_cores=NC, num_subcores=NS)
    return pl.kernel(
        _sc_seg_body,
        out_type=(jax.ShapeDtypeStruct((NPAD, D), jnp.float32),
                  jax.ShapeDtypeStruct((NPAD, D), jnp.float32)),
        mesh=mesh,
        scratch_types=[
            pltpu.VMEM((GRP, CHUNK), jnp.int32),
            pltpu.VMEM((GRP, CHUNK), jnp.int32),
            pltpu.VMEM((1, TAIL), jnp.int32),
            pltpu.VMEM((1, TAIL), jnp.int32),
            pltpu.VMEM((CHUNK, D), jnp.float32),
            pltpu.VMEM((CHUNK, D), jnp.float32),
            pltpu.VMEM((TAIL, D), jnp.float32),
            pltpu.VMEM_SHARED((NPAD, D), jnp.float32),
            pltpu.SemaphoreType.DMA,
            pltpu.SemaphoreType.DMA,
        ],
    )


def _sc_seg(base, diff, srca, dsta, srct, dstt, zeros):
    return _sc_seg_build()(base, diff, srca, dsta, srct, dstt, zeros)


def kernel(x, edge_index, y, pmp_mask,
           W_fr1, W_be1, Wa1, ba1, W_self1, b_self1,
           W_fr2, W_be2, Wa2, ba2, W_self2, b_self2,
           Wc, bc):
    nt = jnp.where(pmp_mask, y.astype(jnp.int32), 2).astype(jnp.int32)[:, None]
    MAIN = NGRP * GRP * CHUNK
    src2 = edge_index[0].astype(jnp.int32).reshape(NS, EPT)
    dst2 = edge_index[1].astype(jnp.int32).reshape(NS, EPT)
    srca = src2[:, :MAIN].reshape(NS, NGRP, GRP, CHUNK)
    dsta = dst2[:, :MAIN].reshape(NS, NGRP, GRP, CHUNK)
    srct = src2[:, MAIN:].reshape(NS, 1, TAIL)
    dstt = dst2[:, MAIN:].reshape(NS, 1, TAIL)
    zeros = jnp.zeros((RPT, D), jnp.float32)

    wa1r = Wa1.reshape(1, D)
    ba1r = jnp.broadcast_to(ba1.reshape(1, 1), (1, D))
    wa2r = Wa2.reshape(1, D)
    ba2r = jnp.broadcast_to(ba2.reshape(1, 1), (1, D))
    bs1r = b_self1.reshape(1, D)
    bs2r = b_self2.reshape(1, D)
    bcr = bc.reshape(1, ODIM)

    base1, diff1, self1, alpha1 = _dense_in(
        x, nt, W_fr1, W_be1, wa1r, ba1r, W_self1, bs1r)
    A1, B1 = _sc_seg(base1, diff1, srca, dsta, srct, dstt, zeros)
    base2, diff2, self2, alpha2 = _dense_mid(
        A1, B1, alpha1, self1, nt, W_fr2, W_be2, wa2r, ba2r, W_self2, bs2r)
    A2, B2 = _sc_seg(base2, diff2, srca, dsta, srct, dstt, zeros)
    return _dense_out(A2, B2, alpha2, self2, Wc, bcr)


# speedup vs baseline: 1.3514x; 1.0503x over previous
import functools

import jax
import jax.numpy as jnp
from jax import lax
from jax.experimental import pallas as pl
from jax.experimental.pallas import tpu as pltpu
from jax.experimental.pallas import tpu_sc as plsc

N = 10000
E = 320000
D = 128
ODIM = 16

NC = 2
NS = 16
EPT = E // NS
CHUNK = 128
GRP = 13
NGRP = 12
TAIL = EPT - NGRP * GRP * CHUNK
NPAD = 10240
RPT = NPAD // NS
RB = 1000


def _node_tables(h, nt, wfr_ref, wbe_ref, war_ref, bar_ref, wself_ref,
                 bself_ref, base_ref, diff_ref, selfo_ref, alpha_ref):
    hfr = jnp.dot(h, wfr_ref[...], preferred_element_type=jnp.float32)
    hbe = jnp.dot(h, wbe_ref[...], preferred_element_type=jnp.float32)
    selfo_ref[...] = (
        jnp.dot(h, wself_ref[...], preferred_element_type=jnp.float32)
        + bself_ref[...])
    a_pre = jnp.sum(h * war_ref[...], axis=1, keepdims=True) + bar_ref[...][:, :1]
    a = jax.nn.sigmoid(a_pre)
    base_ref[...] = jnp.where(nt == 1, hfr, hbe)
    diff_ref[...] = jnp.where(nt == 2, hfr - hbe, jnp.zeros_like(hfr))
    alpha_ref[...] = jnp.broadcast_to(a, (h.shape[0], D))


def _dense_in_body(x_ref, nt_ref, wfr_ref, wbe_ref, war_ref, bar_ref,
                   wself_ref, bself_ref, base_ref, diff_ref, selfo_ref,
                   alpha_ref):
    _node_tables(x_ref[...], nt_ref[...], wfr_ref, wbe_ref, war_ref, bar_ref,
                 wself_ref, bself_ref, base_ref, diff_ref, selfo_ref, alpha_ref)


def _dense_mid_body(a_ref, b_ref, al_ref, so_ref, nt_ref, wfr_ref, wbe_ref,
                    war_ref, bar_ref, wself_ref, bself_ref, base_ref,
                    diff_ref, selfo_ref, alpha_ref):
    h = jnp.maximum(a_ref[...] + al_ref[...] * b_ref[...] + so_ref[...], 0.0)
    _node_tables(h, nt_ref[...], wfr_ref, wbe_ref, war_ref, bar_ref,
                 wself_ref, bself_ref, base_ref, diff_ref, selfo_ref, alpha_ref)


def _dense_out_body(a_ref, b_ref, al_ref, so_ref, wc_ref, bc_ref, out_ref):
    h = jnp.maximum(a_ref[...] + al_ref[...] * b_ref[...] + so_ref[...], 0.0)
    out_ref[...] = (
        jnp.dot(h, wc_ref[...], preferred_element_type=jnp.float32)
        + bc_ref[...])


_rowmap = lambda i: (i, 0)
_fixmap = lambda i: (0, 0)


def _dense_in(x, nt, wfr, wbe, war, bar, wself, bselfr):
    return pl.pallas_call(
        _dense_in_body,
        grid=(N // RB,),
        in_specs=[
            pl.BlockSpec((RB, D), _rowmap),
            pl.BlockSpec((RB, 1), _rowmap),
            pl.BlockSpec((D, D), _fixmap),
            pl.BlockSpec((D, D), _fixmap),
            pl.BlockSpec((1, D), _fixmap),
            pl.BlockSpec((1, D), _fixmap),
            pl.BlockSpec((D, D), _fixmap),
            pl.BlockSpec((1, D), _fixmap),
        ],
        out_specs=[pl.BlockSpec((RB, D), _rowmap)] * 4,
        out_shape=[jax.ShapeDtypeStruct((N, D), jnp.float32)] * 4,
    )(x, nt, wfr, wbe, war, bar, wself, bselfr)


def _dense_mid(a, b, al, so, nt, wfr, wbe, war, bar, wself, bselfr):
    return pl.pallas_call(
        _dense_mid_body,
        grid=(N // RB,),
        in_specs=[
            pl.BlockSpec((RB, D), _rowmap),
            pl.BlockSpec((RB, D), _rowmap),
            pl.BlockSpec((RB, D), _rowmap),
            pl.BlockSpec((RB, D), _rowmap),
            pl.BlockSpec((RB, 1), _rowmap),
            pl.BlockSpec((D, D), _fixmap),
            pl.BlockSpec((D, D), _fixmap),
            pl.BlockSpec((1, D), _fixmap),
            pl.BlockSpec((1, D), _fixmap),
            pl.BlockSpec((D, D), _fixmap),
            pl.BlockSpec((1, D), _fixmap),
        ],
        out_specs=[pl.BlockSpec((RB, D), _rowmap)] * 4,
        out_shape=[jax.ShapeDtypeStruct((N, D), jnp.float32)] * 4,
    )(a, b, al, so, nt, wfr, wbe, war, bar, wself, bselfr)


def _dense_out(a, b, al, so, wc, bcr):
    return pl.pallas_call(
        _dense_out_body,
        grid=(N // RB,),
        in_specs=[
            pl.BlockSpec((RB, D), _rowmap),
            pl.BlockSpec((RB, D), _rowmap),
            pl.BlockSpec((RB, D), _rowmap),
            pl.BlockSpec((RB, D), _rowmap),
            pl.BlockSpec((D, ODIM), _fixmap),
            pl.BlockSpec((1, ODIM), _fixmap),
        ],
        out_specs=pl.BlockSpec((RB, ODIM), _rowmap),
        out_shape=jax.ShapeDtypeStruct((N, ODIM), jnp.float32),
    )(a, b, al, so, wc, bcr)


def _sc_seg_body(base_hbm, diff_hbm, src_hbm, dst_hbm, srct_hbm, dstt_hbm,
                 zeros_hbm, a_hbm, b_hbm,
                 sidxa, didxa, sidxb, didxb, sidxt, didxt,
                 rows0, rows1, rowst, acc, semg0, semg1, semia, semib):
    c = lax.axis_index("c")
    s = lax.axis_index("s")

    pltpu.sync_copy(zeros_hbm, acc.at[pl.ds(s * RPT, RPT)])
    plsc.subcore_barrier()

    def run(tab_hbm):
        def wait_g(rows, semg):
            pltpu.make_async_copy(tab_hbm.at[sidxa.at[0]], rows, semg).wait()

        def wait_idx(sidx, didx, semi):
            pltpu.make_async_copy(src_hbm.at[s, 0], sidx, semi).wait()
            pltpu.make_async_copy(dst_hbm.at[s, 0], didx, semi).wait()

        def stage(g, sidx, didx, semi):
            pltpu.async_copy(src_hbm.at[s, g], sidx, semi)
            pltpu.async_copy(dst_hbm.at[s, g], didx, semi)

        def process(sidx, didx):
            pltpu.async_copy(tab_hbm.at[sidx.at[0]], rows0, semg0)

            @pl.loop(0, GRP - 1, step=2)
            def _(k):
                pltpu.async_copy(tab_hbm.at[sidx.at[k + 1]], rows1, semg1)
                wait_g(rows0, semg0)
                pltpu.sync_copy(rows0, acc.at[didx.at[k]], add=True)
                pltpu.async_copy(tab_hbm.at[sidx.at[k + 2]], rows0, semg0)
                wait_g(rows1, semg1)
                pltpu.sync_copy(rows1, acc.at[didx.at[k + 1]], add=True)

            wait_g(rows0, semg0)
            pltpu.sync_copy(rows0, acc.at[didx.at[GRP - 1]], add=True)

        stage(0, sidxa, didxa, semia)

        @pl.loop(0, NGRP, step=2)
        def _(g):
            stage(g + 1, sidxb, didxb, semib)
            wait_idx(sidxa, didxa, semia)
            process(sidxa, didxa)

            @pl.when(g + 2 < NGRP)
            def _():
                stage(g + 2, sidxa, didxa, semia)

            wait_idx(sidxb, didxb, semib)
            process(sidxb, didxb)

        pltpu.sync_copy(srct_hbm.at[s], sidxt)
        pltpu.sync_copy(dstt_hbm.at[s], didxt)
        pltpu.async_copy(tab_hbm.at[sidxt.at[0]], rowst, semg0).wait()
        pltpu.sync_copy(rowst, acc.at[didxt.at[0]], add=True)

    @pl.when(c == 0)
    def _():
        run(base_hbm)

    @pl.when(c == 1)
    def _():
        run(diff_hbm)

    plsc.subcore_barrier()

    @pl.when(c == 0)
    def _():
        pltpu.sync_copy(acc.at[pl.ds(s * RPT, RPT)],
                        a_hbm.at[pl.ds(s * RPT, RPT)])

    @pl.when(c == 1)
    def _():
        pltpu.sync_copy(acc.at[pl.ds(s * RPT, RPT)],
                        b_hbm.at[pl.ds(s * RPT, RPT)])


@functools.cache
def _sc_seg_build():
    mesh = plsc.VectorSubcoreMesh(core_axis_name="c", subcore_axis_name="s",
                                  num_cores=NC, num_subcores=NS)
    return pl.kernel(
        _sc_seg_body,
        out_type=(jax.ShapeDtypeStruct((NPAD, D), jnp.float32),
                  jax.ShapeDtypeStruct((NPAD, D), jnp.float32)),
        mesh=mesh,
        scratch_types=[
            pltpu.VMEM((GRP, CHUNK), jnp.int32),
            pltpu.VMEM((GRP, CHUNK), jnp.int32),
            pltpu.VMEM((GRP, CHUNK), jnp.int32),
            pltpu.VMEM((GRP, CHUNK), jnp.int32),
            pltpu.VMEM((1, TAIL), jnp.int32),
            pltpu.VMEM((1, TAIL), jnp.int32),
            pltpu.VMEM((CHUNK, D), jnp.float32),
            pltpu.VMEM((CHUNK, D), jnp.float32),
            pltpu.VMEM((TAIL, D), jnp.float32),
            pltpu.VMEM_SHARED((NPAD, D), jnp.float32),
            pltpu.SemaphoreType.DMA,
            pltpu.SemaphoreType.DMA,
            pltpu.SemaphoreType.DMA,
            pltpu.SemaphoreType.DMA,
        ],
    )


def _sc_seg(base, diff, srca, dsta, srct, dstt, zeros):
    return _sc_seg_build()(base, diff, srca, dsta, srct, dstt, zeros)


def kernel(x, edge_index, y, pmp_mask,
           W_fr1, W_be1, Wa1, ba1, W_self1, b_self1,
           W_fr2, W_be2, Wa2, ba2, W_self2, b_self2,
           Wc, bc):
    nt = jnp.where(pmp_mask, y.astype(jnp.int32), 2).astype(jnp.int32)[:, None]
    MAIN = NGRP * GRP * CHUNK
    src2 = edge_index[0].astype(jnp.int32).reshape(NS, EPT)
    dst2 = edge_index[1].astype(jnp.int32).reshape(NS, EPT)
    srca = src2[:, :MAIN].reshape(NS, NGRP, GRP, CHUNK)
    dsta = dst2[:, :MAIN].reshape(NS, NGRP, GRP, CHUNK)
    srct = src2[:, MAIN:].reshape(NS, 1, TAIL)
    dstt = dst2[:, MAIN:].reshape(NS, 1, TAIL)
    zeros = jnp.zeros((RPT, D), jnp.float32)

    wa1r = Wa1.reshape(1, D)
    ba1r = jnp.broadcast_to(ba1.reshape(1, 1), (1, D))
    wa2r = Wa2.reshape(1, D)
    ba2r = jnp.broadcast_to(ba2.reshape(1, 1), (1, D))
    bs1r = b_self1.reshape(1, D)
    bs2r = b_self2.reshape(1, D)
    bcr = bc.reshape(1, ODIM)

    base1, diff1, self1, alpha1 = _dense_in(
        x, nt, W_fr1, W_be1, wa1r, ba1r, W_self1, bs1r)
    A1, B1 = _sc_seg(base1, diff1, srca, dsta, srct, dstt, zeros)
    base2, diff2, self2, alpha2 = _dense_mid(
        A1, B1, alpha1, self1, nt, W_fr2, W_be2, wa2r, ba2r, W_self2, bs2r)
    A2, B2 = _sc_seg(base2, diff2, srca, dsta, srct, dstt, zeros)
    return _dense_out(A2, B2, alpha2, self2, Wc, bcr)


# speedup vs baseline: 1.4158x; 1.0476x over previous
import functools

import jax
import jax.numpy as jnp
from jax import lax
from jax.experimental import pallas as pl
from jax.experimental.pallas import tpu as pltpu
from jax.experimental.pallas import tpu_sc as plsc

N = 10000
E = 320000
D = 128
ODIM = 16

NC = 2
NS = 16
EPT = E // NS
CHUNK = 104
GRP = 12
NGRP = 16
TAIL = EPT - NGRP * GRP * CHUNK
NPAD = 10240
RPT = NPAD // NS
RB = 1000


def _node_tables(h, nt, wfr_ref, wbe_ref, war_ref, bar_ref, wself_ref,
                 bself_ref, base_ref, diff_ref, selfo_ref, alpha_ref):
    hfr = jnp.dot(h, wfr_ref[...], preferred_element_type=jnp.float32)
    hbe = jnp.dot(h, wbe_ref[...], preferred_element_type=jnp.float32)
    selfo_ref[...] = (
        jnp.dot(h, wself_ref[...], preferred_element_type=jnp.float32)
        + bself_ref[...])
    a_pre = jnp.sum(h * war_ref[...], axis=1, keepdims=True) + bar_ref[...][:, :1]
    a = jax.nn.sigmoid(a_pre)
    base_ref[...] = jnp.where(nt == 1, hfr, hbe)
    diff_ref[...] = jnp.where(nt == 2, hfr - hbe, jnp.zeros_like(hfr))
    alpha_ref[...] = jnp.broadcast_to(a, (h.shape[0], D))


def _dense_in_body(x_ref, nt_ref, wfr_ref, wbe_ref, war_ref, bar_ref,
                   wself_ref, bself_ref, base_ref, diff_ref, selfo_ref,
                   alpha_ref):
    _node_tables(x_ref[...], nt_ref[...], wfr_ref, wbe_ref, war_ref, bar_ref,
                 wself_ref, bself_ref, base_ref, diff_ref, selfo_ref, alpha_ref)


def _dense_mid_body(a_ref, b_ref, al_ref, so_ref, nt_ref, wfr_ref, wbe_ref,
                    war_ref, bar_ref, wself_ref, bself_ref, base_ref,
                    diff_ref, selfo_ref, alpha_ref):
    h = jnp.maximum(a_ref[...] + al_ref[...] * b_ref[...] + so_ref[...], 0.0)
    _node_tables(h, nt_ref[...], wfr_ref, wbe_ref, war_ref, bar_ref,
                 wself_ref, bself_ref, base_ref, diff_ref, selfo_ref, alpha_ref)


def _dense_out_body(a_ref, b_ref, al_ref, so_ref, wc_ref, bc_ref, out_ref):
    h = jnp.maximum(a_ref[...] + al_ref[...] * b_ref[...] + so_ref[...], 0.0)
    out_ref[...] = (
        jnp.dot(h, wc_ref[...], preferred_element_type=jnp.float32)
        + bc_ref[...])


_rowmap = lambda i: (i, 0)
_fixmap = lambda i: (0, 0)


def _dense_in(x, nt, wfr, wbe, war, bar, wself, bselfr):
    return pl.pallas_call(
        _dense_in_body,
        grid=(N // RB,),
        in_specs=[
            pl.BlockSpec((RB, D), _rowmap),
            pl.BlockSpec((RB, 1), _rowmap),
            pl.BlockSpec((D, D), _fixmap),
            pl.BlockSpec((D, D), _fixmap),
            pl.BlockSpec((1, D), _fixmap),
            pl.BlockSpec((1, D), _fixmap),
            pl.BlockSpec((D, D), _fixmap),
            pl.BlockSpec((1, D), _fixmap),
        ],
        out_specs=[pl.BlockSpec((RB, D), _rowmap)] * 4,
        out_shape=[jax.ShapeDtypeStruct((N, D), jnp.float32)] * 4,
    )(x, nt, wfr, wbe, war, bar, wself, bselfr)


def _dense_mid(a, b, al, so, nt, wfr, wbe, war, bar, wself, bselfr):
    return pl.pallas_call(
        _dense_mid_body,
        grid=(N // RB,),
        in_specs=[
            pl.BlockSpec((RB, D), _rowmap),
            pl.BlockSpec((RB, D), _rowmap),
            pl.BlockSpec((RB, D), _rowmap),
            pl.BlockSpec((RB, D), _rowmap),
            pl.BlockSpec((RB, 1), _rowmap),
            pl.BlockSpec((D, D), _fixmap),
            pl.BlockSpec((D, D), _fixmap),
            pl.BlockSpec((1, D), _fixmap),
            pl.BlockSpec((1, D), _fixmap),
            pl.BlockSpec((D, D), _fixmap),
            pl.BlockSpec((1, D), _fixmap),
        ],
        out_specs=[pl.BlockSpec((RB, D), _rowmap)] * 4,
        out_shape=[jax.ShapeDtypeStruct((N, D), jnp.float32)] * 4,
    )(a, b, al, so, nt, wfr, wbe, war, bar, wself, bselfr)


def _dense_out(a, b, al, so, wc, bcr):
    return pl.pallas_call(
        _dense_out_body,
        grid=(N // RB,),
        in_specs=[
            pl.BlockSpec((RB, D), _rowmap),
            pl.BlockSpec((RB, D), _rowmap),
            pl.BlockSpec((RB, D), _rowmap),
            pl.BlockSpec((RB, D), _rowmap),
            pl.BlockSpec((D, ODIM), _fixmap),
            pl.BlockSpec((1, ODIM), _fixmap),
        ],
        out_specs=pl.BlockSpec((RB, ODIM), _rowmap),
        out_shape=jax.ShapeDtypeStruct((N, ODIM), jnp.float32),
    )(a, b, al, so, wc, bcr)


def _sc_seg_body(base_hbm, diff_hbm, src_hbm, dst_hbm, srct_hbm, dstt_hbm,
                 zeros_hbm, a_hbm, b_hbm,
                 sidxa, didxa, sidxb, didxb, sidxt, didxt,
                 rows0, rows1, rows2, acc, semg0, semg1, semg2, semia, semib):
    c = lax.axis_index("c")
    s = lax.axis_index("s")

    pltpu.sync_copy(zeros_hbm, acc.at[pl.ds(s * RPT, RPT)])
    plsc.subcore_barrier()

    def run(tab_hbm):
        def wait_g(rows, semg):
            pltpu.make_async_copy(tab_hbm.at[sidxa.at[0]], rows, semg).wait()

        def wait_idx(sidx, didx, semi):
            pltpu.make_async_copy(src_hbm.at[s, 0], sidx, semi).wait()
            pltpu.make_async_copy(dst_hbm.at[s, 0], didx, semi).wait()

        def stage(g, sidx, didx, semi):
            pltpu.async_copy(src_hbm.at[s, g], sidx, semi)
            pltpu.async_copy(dst_hbm.at[s, g], didx, semi)

        def process(sidx, didx):
            bufs = ((rows0, semg0), (rows1, semg1), (rows2, semg2))
            for j, (rows, semg) in enumerate(bufs):
                pltpu.async_copy(tab_hbm.at[sidx.at[j]], rows, semg)

            @pl.loop(0, GRP - 3, step=3)
            def _(k):
                for j, (rows, semg) in enumerate(bufs):
                    wait_g(rows, semg)
                    pltpu.sync_copy(rows, acc.at[didx.at[k + j]], add=True)
                    pltpu.async_copy(tab_hbm.at[sidx.at[k + j + 3]], rows, semg)

            for j, (rows, semg) in enumerate(bufs):
                wait_g(rows, semg)
                pltpu.sync_copy(rows, acc.at[didx.at[GRP - 3 + j]], add=True)

        stage(0, sidxa, didxa, semia)

        @pl.loop(0, NGRP, step=2)
        def _(g):
            stage(g + 1, sidxb, didxb, semib)
            wait_idx(sidxa, didxa, semia)
            process(sidxa, didxa)

            @pl.when(g + 2 < NGRP)
            def _():
                stage(g + 2, sidxa, didxa, semia)

            wait_idx(sidxb, didxb, semib)
            process(sidxb, didxb)

        pltpu.sync_copy(srct_hbm.at[s], sidxt)
        pltpu.sync_copy(dstt_hbm.at[s], didxt)
        pltpu.async_copy(tab_hbm.at[sidxt.at[0]], rows0.at[pl.ds(0, TAIL)],
                         semg0).wait()
        pltpu.sync_copy(rows0.at[pl.ds(0, TAIL)], acc.at[didxt.at[0]], add=True)

    @pl.when(c == 0)
    def _():
        run(base_hbm)

    @pl.when(c == 1)
    def _():
        run(diff_hbm)

    plsc.subcore_barrier()

    @pl.when(c == 0)
    def _():
        pltpu.sync_copy(acc.at[pl.ds(s * RPT, RPT)],
                        a_hbm.at[pl.ds(s * RPT, RPT)])

    @pl.when(c == 1)
    def _():
        pltpu.sync_copy(acc.at[pl.ds(s * RPT, RPT)],
                        b_hbm.at[pl.ds(s * RPT, RPT)])


@functools.cache
def _sc_seg_build():
    mesh = plsc.VectorSubcoreMesh(core_axis_name="c", subcore_axis_name="s",
                                  num_cores=NC, num_subcores=NS)
    return pl.kernel(
        _sc_seg_body,
        out_type=(jax.ShapeDtypeStruct((NPAD, D), jnp.float32),
                  jax.ShapeDtypeStruct((NPAD, D), jnp.float32)),
        mesh=mesh,
        scratch_types=[
            pltpu.VMEM((GRP, CHUNK), jnp.int32),
            pltpu.VMEM((GRP, CHUNK), jnp.int32),
            pltpu.VMEM((GRP, CHUNK), jnp.int32),
            pltpu.VMEM((GRP, CHUNK), jnp.int32),
            pltpu.VMEM((1, TAIL), jnp.int32),
            pltpu.VMEM((1, TAIL), jnp.int32),
            pltpu.VMEM((CHUNK, D), jnp.float32),
            pltpu.VMEM((CHUNK, D), jnp.float32),
            pltpu.VMEM((CHUNK, D), jnp.float32),
            pltpu.VMEM_SHARED((NPAD, D), jnp.float32),
            pltpu.SemaphoreType.DMA,
            pltpu.SemaphoreType.DMA,
            pltpu.SemaphoreType.DMA,
            pltpu.SemaphoreType.DMA,
            pltpu.SemaphoreType.DMA,
        ],
    )


def _sc_seg(base, diff, srca, dsta, srct, dstt, zeros):
    return _sc_seg_build()(base, diff, srca, dsta, srct, dstt, zeros)


def kernel(x, edge_index, y, pmp_mask,
           W_fr1, W_be1, Wa1, ba1, W_self1, b_self1,
           W_fr2, W_be2, Wa2, ba2, W_self2, b_self2,
           Wc, bc):
    nt = jnp.where(pmp_mask, y.astype(jnp.int32), 2).astype(jnp.int32)[:, None]
    MAIN = NGRP * GRP * CHUNK
    src2 = edge_index[0].astype(jnp.int32).reshape(NS, EPT)
    dst2 = edge_index[1].astype(jnp.int32).reshape(NS, EPT)
    srca = src2[:, :MAIN].reshape(NS, NGRP, GRP, CHUNK)
    dsta = dst2[:, :MAIN].reshape(NS, NGRP, GRP, CHUNK)
    srct = src2[:, MAIN:].reshape(NS, 1, TAIL)
    dstt = dst2[:, MAIN:].reshape(NS, 1, TAIL)
    zeros = jnp.zeros((RPT, D), jnp.float32)

    wa1r = Wa1.reshape(1, D)
    ba1r = jnp.broadcast_to(ba1.reshape(1, 1), (1, D))
    wa2r = Wa2.reshape(1, D)
    ba2r = jnp.broadcast_to(ba2.reshape(1, 1), (1, D))
    bs1r = b_self1.reshape(1, D)
    bs2r = b_self2.reshape(1, D)
    bcr = bc.reshape(1, ODIM)

    base1, diff1, self1, alpha1 = _dense_in(
        x, nt, W_fr1, W_be1, wa1r, ba1r, W_self1, bs1r)
    A1, B1 = _sc_seg(base1, diff1, srca, dsta, srct, dstt, zeros)
    base2, diff2, self2, alpha2 = _dense_mid(
        A1, B1, alpha1, self1, nt, W_fr2, W_be2, wa2r, ba2r, W_self2, bs2r)
    A2, B2 = _sc_seg(base2, diff2, srca, dsta, srct, dstt, zeros)
    return _dense_out(A2, B2, alpha2, self2, Wc, bcr)


# speedup vs baseline: 1.4375x; 1.0153x over previous
import functools

import jax
import jax.numpy as jnp
from jax import lax
from jax.experimental import pallas as pl
from jax.experimental.pallas import tpu as pltpu
from jax.experimental.pallas import tpu_sc as plsc

N = 10000
E = 320000
D = 128
ODIM = 16

NC = 2
NS = 16
EPT = E // NS
CHUNK = 104
GRP = 12
NGRP = 16
TAIL = EPT - NGRP * GRP * CHUNK
NPAD = 10240
RPT = NPAD // NS
RB = 2000


def _node_tables(h, nt, wfr_ref, wbe_ref, war_ref, bar_ref, wself_ref,
                 bself_ref, base_ref, diff_ref, selfo_ref, alpha_ref):
    hfr = jnp.dot(h, wfr_ref[...], preferred_element_type=jnp.float32)
    hbe = jnp.dot(h, wbe_ref[...], preferred_element_type=jnp.float32)
    selfo_ref[...] = (
        jnp.dot(h, wself_ref[...], preferred_element_type=jnp.float32)
        + bself_ref[...])
    a_pre = jnp.sum(h * war_ref[...], axis=1, keepdims=True) + bar_ref[...][:, :1]
    a = jax.nn.sigmoid(a_pre)
    base_ref[...] = jnp.where(nt == 1, hfr, hbe)
    diff_ref[...] = jnp.where(nt == 2, hfr - hbe, jnp.zeros_like(hfr))
    alpha_ref[...] = jnp.broadcast_to(a, (h.shape[0], D))


def _dense_in_body(x_ref, nt_ref, wfr_ref, wbe_ref, war_ref, bar_ref,
                   wself_ref, bself_ref, base_ref, diff_ref, selfo_ref,
                   alpha_ref):
    _node_tables(x_ref[...], nt_ref[...], wfr_ref, wbe_ref, war_ref, bar_ref,
                 wself_ref, bself_ref, base_ref, diff_ref, selfo_ref, alpha_ref)


def _dense_mid_body(a_ref, b_ref, al_ref, so_ref, nt_ref, wfr_ref, wbe_ref,
                    war_ref, bar_ref, wself_ref, bself_ref, base_ref,
                    diff_ref, selfo_ref, alpha_ref):
    h = jnp.maximum(a_ref[...] + al_ref[...] * b_ref[...] + so_ref[...], 0.0)
    _node_tables(h, nt_ref[...], wfr_ref, wbe_ref, war_ref, bar_ref,
                 wself_ref, bself_ref, base_ref, diff_ref, selfo_ref, alpha_ref)


def _dense_out_body(a_ref, b_ref, al_ref, so_ref, wc_ref, bc_ref, out_ref):
    h = jnp.maximum(a_ref[...] + al_ref[...] * b_ref[...] + so_ref[...], 0.0)
    out_ref[...] = (
        jnp.dot(h, wc_ref[...], preferred_element_type=jnp.float32)
        + bc_ref[...])


_rowmap = lambda i: (i, 0)
_fixmap = lambda i: (0, 0)


def _dense_in(x, nt, wfr, wbe, war, bar, wself, bselfr):
    return pl.pallas_call(
        _dense_in_body,
        grid=(N // RB,),
        in_specs=[
            pl.BlockSpec((RB, D), _rowmap),
            pl.BlockSpec((RB, 1), _rowmap),
            pl.BlockSpec((D, D), _fixmap),
            pl.BlockSpec((D, D), _fixmap),
            pl.BlockSpec((1, D), _fixmap),
            pl.BlockSpec((1, D), _fixmap),
            pl.BlockSpec((D, D), _fixmap),
            pl.BlockSpec((1, D), _fixmap),
        ],
        out_specs=[pl.BlockSpec((RB, D), _rowmap)] * 4,
        out_shape=[jax.ShapeDtypeStruct((N, D), jnp.float32)] * 4,
    )(x, nt, wfr, wbe, war, bar, wself, bselfr)


def _dense_mid(a, b, al, so, nt, wfr, wbe, war, bar, wself, bselfr):
    return pl.pallas_call(
        _dense_mid_body,
        grid=(N // RB,),
        in_specs=[
            pl.BlockSpec((RB, D), _rowmap),
            pl.BlockSpec((RB, D), _rowmap),
            pl.BlockSpec((RB, D), _rowmap),
            pl.BlockSpec((RB, D), _rowmap),
            pl.BlockSpec((RB, 1), _rowmap),
            pl.BlockSpec((D, D), _fixmap),
            pl.BlockSpec((D, D), _fixmap),
            pl.BlockSpec((1, D), _fixmap),
            pl.BlockSpec((1, D), _fixmap),
            pl.BlockSpec((D, D), _fixmap),
            pl.BlockSpec((1, D), _fixmap),
        ],
        out_specs=[pl.BlockSpec((RB, D), _rowmap)] * 4,
        out_shape=[jax.ShapeDtypeStruct((N, D), jnp.float32)] * 4,
    )(a, b, al, so, nt, wfr, wbe, war, bar, wself, bselfr)


def _dense_out(a, b, al, so, wc, bcr):
    return pl.pallas_call(
        _dense_out_body,
        grid=(N // RB,),
        in_specs=[
            pl.BlockSpec((RB, D), _rowmap),
            pl.BlockSpec((RB, D), _rowmap),
            pl.BlockSpec((RB, D), _rowmap),
            pl.BlockSpec((RB, D), _rowmap),
            pl.BlockSpec((D, ODIM), _fixmap),
            pl.BlockSpec((1, ODIM), _fixmap),
        ],
        out_specs=pl.BlockSpec((RB, ODIM), _rowmap),
        out_shape=jax.ShapeDtypeStruct((N, ODIM), jnp.float32),
    )(a, b, al, so, wc, bcr)


def _sc_seg_body(base_hbm, diff_hbm, src_hbm, dst_hbm, srct_hbm, dstt_hbm,
                 zeros_hbm, a_hbm, b_hbm,
                 sidxa, didxa, sidxb, didxb, sidxt, didxt,
                 rows0, rows1, rows2, acc, semg0, semg1, semg2, semia, semib):
    c = lax.axis_index("c")
    s = lax.axis_index("s")

    pltpu.sync_copy(zeros_hbm, acc.at[pl.ds(s * RPT, RPT)])
    plsc.subcore_barrier()

    def run(tab_hbm):
        def wait_g(rows, semg):
            pltpu.make_async_copy(tab_hbm.at[sidxa.at[0]], rows, semg).wait()

        def wait_idx(sidx, didx, semi):
            pltpu.make_async_copy(src_hbm.at[s, 0], sidx, semi).wait()
            pltpu.make_async_copy(dst_hbm.at[s, 0], didx, semi).wait()

        def stage(g, sidx, didx, semi):
            pltpu.async_copy(src_hbm.at[s, g], sidx, semi)
            pltpu.async_copy(dst_hbm.at[s, g], didx, semi)

        def process(sidx, didx):
            bufs = ((rows0, semg0), (rows1, semg1), (rows2, semg2))
            for j, (rows, semg) in enumerate(bufs):
                pltpu.async_copy(tab_hbm.at[sidx.at[j]], rows, semg)

            @pl.loop(0, GRP - 3, step=3)
            def _(k):
                for j, (rows, semg) in enumerate(bufs):
                    wait_g(rows, semg)
                    pltpu.sync_copy(rows, acc.at[didx.at[k + j]], add=True)
                    pltpu.async_copy(tab_hbm.at[sidx.at[k + j + 3]], rows, semg)

            for j, (rows, semg) in enumerate(bufs):
                wait_g(rows, semg)
                pltpu.sync_copy(rows, acc.at[didx.at[GRP - 3 + j]], add=True)

        stage(0, sidxa, didxa, semia)

        @pl.loop(0, NGRP, step=2)
        def _(g):
            stage(g + 1, sidxb, didxb, semib)
            wait_idx(sidxa, didxa, semia)
            process(sidxa, didxa)

            @pl.when(g + 2 < NGRP)
            def _():
                stage(g + 2, sidxa, didxa, semia)

            wait_idx(sidxb, didxb, semib)
            process(sidxb, didxb)

        pltpu.sync_copy(srct_hbm.at[s], sidxt)
        pltpu.sync_copy(dstt_hbm.at[s], didxt)
        pltpu.async_copy(tab_hbm.at[sidxt.at[0]], rows0.at[pl.ds(0, TAIL)],
                         semg0).wait()
        pltpu.sync_copy(rows0.at[pl.ds(0, TAIL)], acc.at[didxt.at[0]], add=True)

    @pl.when(c == 0)
    def _():
        run(base_hbm)

    @pl.when(c == 1)
    def _():
        run(diff_hbm)

    plsc.subcore_barrier()

    @pl.when(c == 0)
    def _():
        pltpu.sync_copy(acc.at[pl.ds(s * RPT, RPT)],
                        a_hbm.at[pl.ds(s * RPT, RPT)])

    @pl.when(c == 1)
    def _():
        pltpu.sync_copy(acc.at[pl.ds(s * RPT, RPT)],
                        b_hbm.at[pl.ds(s * RPT, RPT)])


@functools.cache
def _sc_seg_build():
    mesh = plsc.VectorSubcoreMesh(core_axis_name="c", subcore_axis_name="s",
                                  num_cores=NC, num_subcores=NS)
    return pl.kernel(
        _sc_seg_body,
        out_type=(jax.ShapeDtypeStruct((NPAD, D), jnp.float32),
                  jax.ShapeDtypeStruct((NPAD, D), jnp.float32)),
        mesh=mesh,
        scratch_types=[
            pltpu.VMEM((GRP, CHUNK), jnp.int32),
            pltpu.VMEM((GRP, CHUNK), jnp.int32),
            pltpu.VMEM((GRP, CHUNK), jnp.int32),
            pltpu.VMEM((GRP, CHUNK), jnp.int32),
            pltpu.VMEM((1, TAIL), jnp.int32),
            pltpu.VMEM((1, TAIL), jnp.int32),
            pltpu.VMEM((CHUNK, D), jnp.float32),
            pltpu.VMEM((CHUNK, D), jnp.float32),
            pltpu.VMEM((CHUNK, D), jnp.float32),
            pltpu.VMEM_SHARED((NPAD, D), jnp.float32),
            pltpu.SemaphoreType.DMA,
            pltpu.SemaphoreType.DMA,
            pltpu.SemaphoreType.DMA,
            pltpu.SemaphoreType.DMA,
            pltpu.SemaphoreType.DMA,
        ],
    )


def _sc_seg(base, diff, srca, dsta, srct, dstt, zeros):
    return _sc_seg_build()(base, diff, srca, dsta, srct, dstt, zeros)


def kernel(x, edge_index, y, pmp_mask,
           W_fr1, W_be1, Wa1, ba1, W_self1, b_self1,
           W_fr2, W_be2, Wa2, ba2, W_self2, b_self2,
           Wc, bc):
    nt = jnp.where(pmp_mask, y.astype(jnp.int32), 2).astype(jnp.int32)[:, None]
    MAIN = NGRP * GRP * CHUNK
    src2 = edge_index[0].astype(jnp.int32).reshape(NS, EPT)
    dst2 = edge_index[1].astype(jnp.int32).reshape(NS, EPT)
    srca = src2[:, :MAIN].reshape(NS, NGRP, GRP, CHUNK)
    dsta = dst2[:, :MAIN].reshape(NS, NGRP, GRP, CHUNK)
    srct = src2[:, MAIN:].reshape(NS, 1, TAIL)
    dstt = dst2[:, MAIN:].reshape(NS, 1, TAIL)
    zeros = jnp.zeros((RPT, D), jnp.float32)

    wa1r = Wa1.reshape(1, D)
    ba1r = jnp.broadcast_to(ba1.reshape(1, 1), (1, D))
    wa2r = Wa2.reshape(1, D)
    ba2r = jnp.broadcast_to(ba2.reshape(1, 1), (1, D))
    bs1r = b_self1.reshape(1, D)
    bs2r = b_self2.reshape(1, D)
    bcr = bc.reshape(1, ODIM)

    base1, diff1, self1, alpha1 = _dense_in(
        x, nt, W_fr1, W_be1, wa1r, ba1r, W_self1, bs1r)
    A1, B1 = _sc_seg(base1, diff1, srca, dsta, srct, dstt, zeros)
    base2, diff2, self2, alpha2 = _dense_mid(
        A1, B1, alpha1, self1, nt, W_fr2, W_be2, wa2r, ba2r, W_self2, bs2r)
    A2, B2 = _sc_seg(base2, diff2, srca, dsta, srct, dstt, zeros)
    return _dense_out(A2, B2, alpha2, self2, Wc, bcr)
